# baseline (device time: 203995 ns/iter reference)
import jax
import jax.numpy as jnp
from jax import lax
from jax.experimental import pallas as pl
from jax.experimental.pallas import tpu as pltpu

N_DEV = 16


def kernel(x, w_mat, scale_x, scale_w):
    m_per, k = x.shape
    n = w_mat.shape[1]

    def body(x_ref, w_ref, sx_ref, sw_ref, out_ref, comm_ref,
             send_sems, recv_sems):
        my = lax.axis_index("i")
        left = lax.rem(my + N_DEV - 1, N_DEV)
        right = lax.rem(my + 1, N_DEV)

        comm_ref[0] = x_ref[...]

        barrier_sem = pltpu.get_barrier_semaphore()
        for nbr in (left, right):
            pl.semaphore_signal(
                barrier_sem, inc=1,
                device_id=(nbr,), device_id_type=pl.DeviceIdType.MESH,
            )
        pl.semaphore_wait(barrier_sem, 2)

        scale = sx_ref[0] * sw_ref[0]

        def gemm_slot(slot):
            o = lax.rem(my + N_DEV - slot, N_DEV)
            acc = lax.dot_general(
                comm_ref[slot], w_ref[...],
                (((1,), (0,)), ((), ())),
                preferred_element_type=jnp.int32,
            )
            y = acc.astype(jnp.float32) * scale
            out_ref[pl.ds(o * m_per, m_per), :] = y / (1.0 + jnp.exp(-y))

        for h in range(N_DEV - 1):
            rdma = pltpu.make_async_remote_copy(
                src_ref=comm_ref.at[h],
                dst_ref=comm_ref.at[h + 1],
                send_sem=send_sems.at[h],
                recv_sem=recv_sems.at[h],
                device_id=(right,),
                device_id_type=pl.DeviceIdType.MESH,
            )
            rdma.start()
            gemm_slot(h)
            rdma.wait()
        gemm_slot(N_DEV - 1)

    return pl.pallas_call(
        body,
        out_shape=jax.ShapeDtypeStruct((N_DEV * m_per, n), jnp.float32),
        in_specs=[
            pl.BlockSpec(memory_space=pltpu.VMEM),
            pl.BlockSpec(memory_space=pltpu.VMEM),
            pl.BlockSpec(memory_space=pltpu.SMEM),
            pl.BlockSpec(memory_space=pltpu.SMEM),
        ],
        out_specs=pl.BlockSpec(memory_space=pltpu.VMEM),
        scratch_shapes=[
            pltpu.VMEM((N_DEV, m_per, k), jnp.int8),
            pltpu.SemaphoreType.DMA((N_DEV - 1,)),
            pltpu.SemaphoreType.DMA((N_DEV - 1,)),
        ],
        compiler_params=pltpu.CompilerParams(collective_id=0),
    )(x, w_mat, scale_x, scale_w)


# device time: 119245 ns/iter; 1.7107x vs baseline; 1.7107x over previous
import jax
import jax.numpy as jnp
from jax import lax
from jax.experimental import pallas as pl
from jax.experimental.pallas import tpu as pltpu

N_DEV = 16
H = N_DEV // 2


def kernel(x, w_mat, scale_x, scale_w):
    m_per, k = x.shape
    n = w_mat.shape[1]
    half = m_per // 2

    def body(x_ref, w_ref, sx_ref, sw_ref, out_ref, cr, cl,
             r_send, r_recv, l_send, l_recv):
        my = lax.axis_index("i")
        left = lax.rem(my + N_DEV - 1, N_DEV)
        right = lax.rem(my + 1, N_DEV)

        cr[0] = x_ref[...]
        cl[0] = x_ref[...]

        barrier_sem = pltpu.get_barrier_semaphore()
        for nbr in (left, right):
            pl.semaphore_signal(
                barrier_sem, inc=1,
                device_id=(nbr,), device_id_type=pl.DeviceIdType.MESH,
            )
        pl.semaphore_wait(barrier_sem, 2)

        scale = sx_ref[0] * sw_ref[0]

        def gemm(chunk, o):
            acc = lax.dot_general(
                chunk, w_ref[...],
                (((1,), (0,)), ((), ())),
                preferred_element_type=jnp.int32,
            )
            y = acc.astype(jnp.float32) * scale
            out_ref[pl.ds(o * m_per, m_per), :] = y / (1.0 + jnp.exp(-y))

        for h in range(1, H + 1):
            if h < H:
                rr = pltpu.make_async_remote_copy(
                    src_ref=cr.at[h - 1], dst_ref=cr.at[h],
                    send_sem=r_send.at[h - 1], recv_sem=r_recv.at[h - 1],
                    device_id=(right,), device_id_type=pl.DeviceIdType.MESH,
                )
                rl = pltpu.make_async_remote_copy(
                    src_ref=cl.at[h - 1], dst_ref=cl.at[h],
                    send_sem=l_send.at[h - 1], recv_sem=l_recv.at[h - 1],
                    device_id=(left,), device_id_type=pl.DeviceIdType.MESH,
                )
            else:
                rr = pltpu.make_async_remote_copy(
                    src_ref=cr.at[h - 1, pl.ds(0, half)],
                    dst_ref=cr.at[h, pl.ds(0, half)],
                    send_sem=r_send.at[h - 1], recv_sem=r_recv.at[h - 1],
                    device_id=(right,), device_id_type=pl.DeviceIdType.MESH,
                )
                rl = pltpu.make_async_remote_copy(
                    src_ref=cl.at[h - 1, pl.ds(half, half)],
                    dst_ref=cr.at[h, pl.ds(half, half)],
                    send_sem=l_send.at[h - 1], recv_sem=l_recv.at[h - 1],
                    device_id=(left,), device_id_type=pl.DeviceIdType.MESH,
                )
            rr.start()
            rl.start()
            if h == 1:
                gemm(cr[0], my)
            else:
                gemm(cr[h - 1], lax.rem(my + N_DEV - (h - 1), N_DEV))
                gemm(cl[h - 1], lax.rem(my + h - 1, N_DEV))
            rr.wait()
            rl.wait()

        gemm(cr[H], lax.rem(my + H, N_DEV))

    return pl.pallas_call(
        body,
        out_shape=jax.ShapeDtypeStruct((N_DEV * m_per, n), jnp.float32),
        in_specs=[
            pl.BlockSpec(memory_space=pltpu.VMEM),
            pl.BlockSpec(memory_space=pltpu.VMEM),
            pl.BlockSpec(memory_space=pltpu.SMEM),
            pl.BlockSpec(memory_space=pltpu.SMEM),
        ],
        out_specs=pl.BlockSpec(memory_space=pltpu.VMEM),
        scratch_shapes=[
            pltpu.VMEM((H + 1, m_per, k), jnp.int8),
            pltpu.VMEM((H + 1, m_per, k), jnp.int8),
            pltpu.SemaphoreType.DMA((H,)),
            pltpu.SemaphoreType.DMA((H,)),
            pltpu.SemaphoreType.DMA((H,)),
            pltpu.SemaphoreType.DMA((H,)),
        ],
        compiler_params=pltpu.CompilerParams(collective_id=0),
    )(x, w_mat, scale_x, scale_w)


# device time: 98172 ns/iter; 2.0779x vs baseline; 1.2147x over previous
import jax
import jax.numpy as jnp
from jax import lax
from jax.experimental import pallas as pl
from jax.experimental.pallas import tpu as pltpu

N_DEV = 16
H = N_DEV // 2
S = 2

RING = [0, 1, 5, 9, 13, 14, 10, 6, 2, 3, 7, 11, 15, 12, 8, 4]
POS = [RING.index(m) for m in range(N_DEV)]


def kernel(x, w_mat, scale_x, scale_w):
    m_per, k = x.shape
    n = w_mat.shape[1]
    seg = m_per // S

    def body(x_ref, w_ref, sx_ref, sw_ref, ring_ref, pos_ref, out_ref,
             cr, cl, r_send, r_recv, l_send, l_recv):
        my = lax.axis_index("i")

        p = pos_ref[my]
        right = ring_ref[lax.rem(p + 1, N_DEV)]
        left = ring_ref[lax.rem(p + N_DEV - 1, N_DEV)]

        cr[0] = x_ref[...]
        cl[0] = x_ref[...]

        barrier_sem = pltpu.get_barrier_semaphore()
        for nbr in (left, right):
            pl.semaphore_signal(
                barrier_sem, inc=1,
                device_id=(nbr,), device_id_type=pl.DeviceIdType.MESH,
            )
        pl.semaphore_wait(barrier_sem, 2)

        scale = sx_ref[0] * sw_ref[0]

        def gemm(chunk, o):
            acc = lax.dot_general(
                chunk, w_ref[...],
                (((1,), (0,)), ((), ())),
                preferred_element_type=jnp.int32,
            )
            y = acc.astype(jnp.float32) * scale
            out_ref[pl.ds(o * m_per, m_per), :] = y / (1.0 + jnp.exp(-y))

        rr, rl = {}, {}
        for h in range(1, H + 1):
            for s in range(S):
                if h < H or s == 0:
                    rr[(h, s)] = pltpu.make_async_remote_copy(
                        src_ref=cr.at[h - 1, pl.ds(s * seg, seg)],
                        dst_ref=cr.at[h, pl.ds(s * seg, seg)],
                        send_sem=r_send.at[h - 1, s],
                        recv_sem=r_recv.at[h - 1, s],
                        device_id=(right,),
                        device_id_type=pl.DeviceIdType.MESH,
                    )
                if h < H or s == 1:
                    dst = cr if h == H else cl
                    rl[(h, s)] = pltpu.make_async_remote_copy(
                        src_ref=cl.at[h - 1, pl.ds(s * seg, seg)],
                        dst_ref=dst.at[h, pl.ds(s * seg, seg)],
                        send_sem=l_send.at[h - 1, s],
                        recv_sem=l_recv.at[h - 1, s],
                        device_id=(left,),
                        device_id_type=pl.DeviceIdType.MESH,
                    )

        for s in range(S):
            rr[(1, s)].start()
            rl[(1, s)].start()
        gemm(cr[0], my)

        for h in range(2, H + 1):
            for s in range(S):
                rr[(h - 1, s)].wait_recv()
                if (h, s) in rr:
                    rr[(h, s)].start()
                rl[(h - 1, s)].wait_recv()
                if (h, s) in rl:
                    rl[(h, s)].start()
            gemm(cr[h - 1], ring_ref[lax.rem(p + N_DEV - (h - 1), N_DEV)])
            gemm(cl[h - 1], ring_ref[lax.rem(p + h - 1, N_DEV)])

        rr[(H, 0)].wait_recv()
        rl[(H, 1)].wait_recv()
        gemm(cr[H], ring_ref[lax.rem(p + H, N_DEV)])

        for d in list(rr.values()) + list(rl.values()):
            d.wait_send()

    return pl.pallas_call(
        body,
        out_shape=jax.ShapeDtypeStruct((N_DEV * m_per, n), jnp.float32),
        in_specs=[
            pl.BlockSpec(memory_space=pltpu.VMEM),
            pl.BlockSpec(memory_space=pltpu.VMEM),
            pl.BlockSpec(memory_space=pltpu.SMEM),
            pl.BlockSpec(memory_space=pltpu.SMEM),
            pl.BlockSpec(memory_space=pltpu.SMEM),
            pl.BlockSpec(memory_space=pltpu.SMEM),
        ],
        out_specs=pl.BlockSpec(memory_space=pltpu.VMEM),
        scratch_shapes=[
            pltpu.VMEM((H + 1, m_per, k), jnp.int8),
            pltpu.VMEM((H + 1, m_per, k), jnp.int8),
            pltpu.SemaphoreType.DMA((H, S)),
            pltpu.SemaphoreType.DMA((H, S)),
            pltpu.SemaphoreType.DMA((H, S)),
            pltpu.SemaphoreType.DMA((H, S)),
        ],
        compiler_params=pltpu.CompilerParams(collective_id=0),
    )(x, w_mat, scale_x, scale_w,
      jnp.array(RING, jnp.int32), jnp.array(POS, jnp.int32))


# device time: 96475 ns/iter; 2.1145x vs baseline; 1.0176x over previous
import jax
import jax.numpy as jnp
from jax import lax
from jax.experimental import pallas as pl
from jax.experimental.pallas import tpu as pltpu

N_DEV = 16
H = N_DEV // 2
S = 4

RING = [0, 1, 5, 9, 13, 14, 10, 6, 2, 3, 7, 11, 15, 12, 8, 4]
POS = [RING.index(m) for m in range(N_DEV)]


def kernel(x, w_mat, scale_x, scale_w):
    m_per, k = x.shape
    n = w_mat.shape[1]
    seg = m_per // S

    def body(x_ref, w_ref, sx_ref, sw_ref, ring_ref, pos_ref, out_ref,
             cr, cl, r_send, r_recv, l_send, l_recv):
        my = lax.axis_index("i")

        p = pos_ref[my]
        right = ring_ref[lax.rem(p + 1, N_DEV)]
        left = ring_ref[lax.rem(p + N_DEV - 1, N_DEV)]

        barrier_sem = pltpu.get_barrier_semaphore()
        for nbr in (left, right):
            pl.semaphore_signal(
                barrier_sem, inc=1,
                device_id=(nbr,), device_id_type=pl.DeviceIdType.MESH,
            )
        pl.semaphore_wait(barrier_sem, 2)

        scale = sx_ref[0] * sw_ref[0]

        def gemm(chunk, o):
            acc = lax.dot_general(
                chunk, w_ref[...],
                (((1,), (0,)), ((), ())),
                preferred_element_type=jnp.int32,
            )
            y = acc.astype(jnp.float32) * scale
            out_ref[pl.ds(o * m_per, m_per), :] = y / (1.0 + jnp.exp(-y))

        rr, rl = {}, {}
        for h in range(1, H + 1):
            for s in range(S):
                if h < H or s < S // 2:
                    src = x_ref if h == 1 else cr.at[h - 1]
                    rr[(h, s)] = pltpu.make_async_remote_copy(
                        src_ref=src.at[pl.ds(s * seg, seg)],
                        dst_ref=cr.at[h, pl.ds(s * seg, seg)],
                        send_sem=r_send.at[h - 1, s],
                        recv_sem=r_recv.at[h - 1, s],
                        device_id=(right,),
                        device_id_type=pl.DeviceIdType.MESH,
                    )
                if h < H or s >= S // 2:
                    src = x_ref if h == 1 else cl.at[h - 1]
                    dst = cr if h == H else cl
                    rl[(h, s)] = pltpu.make_async_remote_copy(
                        src_ref=src.at[pl.ds(s * seg, seg)],
                        dst_ref=dst.at[h, pl.ds(s * seg, seg)],
                        send_sem=l_send.at[h - 1, s],
                        recv_sem=l_recv.at[h - 1, s],
                        device_id=(left,),
                        device_id_type=pl.DeviceIdType.MESH,
                    )

        for s in range(S):
            rr[(1, s)].start()
            rl[(1, s)].start()
        gemm(x_ref[...], my)

        for h in range(2, H + 1):
            for s in range(S):
                rr[(h - 1, s)].wait_recv()
                if (h, s) in rr:
                    rr[(h, s)].start()
                rl[(h - 1, s)].wait_recv()
                if (h, s) in rl:
                    rl[(h, s)].start()
            gemm(cr[h - 1], ring_ref[lax.rem(p + N_DEV - (h - 1), N_DEV)])
            gemm(cl[h - 1], ring_ref[lax.rem(p + h - 1, N_DEV)])

        for s in range(S // 2):
            rr[(H, s)].wait_recv()
        for s in range(S // 2, S):
            rl[(H, s)].wait_recv()
        gemm(cr[H], ring_ref[lax.rem(p + H, N_DEV)])

        for d in list(rr.values()) + list(rl.values()):
            d.wait_send()

    return pl.pallas_call(
        body,
        out_shape=jax.ShapeDtypeStruct((N_DEV * m_per, n), jnp.float32),
        in_specs=[
            pl.BlockSpec(memory_space=pltpu.VMEM),
            pl.BlockSpec(memory_space=pltpu.VMEM),
            pl.BlockSpec(memory_space=pltpu.SMEM),
            pl.BlockSpec(memory_space=pltpu.SMEM),
            pl.BlockSpec(memory_space=pltpu.SMEM),
            pl.BlockSpec(memory_space=pltpu.SMEM),
        ],
        out_specs=pl.BlockSpec(memory_space=pltpu.VMEM),
        scratch_shapes=[
            pltpu.VMEM((H + 1, m_per, k), jnp.int8),
            pltpu.VMEM((H + 1, m_per, k), jnp.int8),
            pltpu.SemaphoreType.DMA((H, S)),
            pltpu.SemaphoreType.DMA((H, S)),
            pltpu.SemaphoreType.DMA((H, S)),
            pltpu.SemaphoreType.DMA((H, S)),
        ],
        compiler_params=pltpu.CompilerParams(collective_id=0),
    )(x, w_mat, scale_x, scale_w,
      jnp.array(RING, jnp.int32), jnp.array(POS, jnp.int32))
